# baseline (speedup 1.0000x reference)
import functools

import jax
import jax.numpy as jnp
import numpy as np
from jax import lax
from jax.experimental import pallas as pl
from jax.experimental.pallas import tpu as pltpu
from jax.experimental.pallas import tpu_sc as plsc

N = 10000
E = 160000
F = 256
HALF = 128
B = 64
NS = 16
EPT = E // NS
K = 80
NCH = EPT // K
ZK = 80
NZC = N // ZK
INV_SQRT = float(1.0 / np.sqrt(1.0 + 1e-5))


BLK = 25


def _sc_agg_body(hL, hR, pk3, outL, outR,
                 pkblk, s0, s1, s2, s3, d0, d1, d2, d3,
                 g0, g1, g2, g3, acc,
                 sg0, sg1, sg2, sg3, ss0, ss1, ss2, ss3):
    g = (g0, g1, g2, g3)
    sv = (s0, s1, s2, s3)
    dv = (d0, d1, d2, d3)
    sg = (sg0, sg1, sg2, sg3)
    ss = (ss0, ss1, ss2, ss3)
    c = lax.axis_index("c")
    s = lax.axis_index("s")

    def stage_block(jn):
        pltpu.sync_copy(pk3.at[s, pl.ds(jn, BLK)], pkblk)

    def unpack(jn, b):
        r = jn % BLK
        for t in range(K // 16):
            v = pkblk[r, pl.ds(t * 16, 16)]
            sv[b][pl.ds(t * 16, 16)] = lax.shift_right_logical(v, 14)
            dv[b][pl.ds(t * 16, 16)] = lax.bitwise_and(v, 16383)

    def init_from(table):
        def zchunk(kk, carry):
            t = s + kk * NS
            @pl.when(t < NZC)
            def _():
                pltpu.sync_copy(table.at[pl.ds(t * ZK, ZK)],
                                acc.at[pl.ds(t * ZK, ZK)])
            return carry
        lax.fori_loop(0, (NZC + NS - 1) // NS, zchunk, 0)

    @pl.when(c == 0)
    def _():
        init_from(hL)

    @pl.when(c == 1)
    def _():
        init_from(hR)

    plsc.subcore_barrier()

    def run(table):
        stage_block(0)
        for b in range(4):
            unpack(b, b)
            pltpu.async_copy(table.at[sv[b]], g[b], sg[b])
        def body(jj, carry):
            j0 = 4 * jj
            for b in range(4):
                j = j0 + b
                pltpu.make_async_copy(table.at[sv[b]], g[b], sg[b]).wait()
                pltpu.async_copy(g[b], acc.at[dv[b]], ss[b], add=True)
                jn = j + 4
                @pl.when(jn < NCH)
                def _(b=b, jn=jn):
                    pltpu.make_async_copy(g[b], acc.at[dv[b]], ss[b]).wait()
                    @pl.when(jn % BLK == 0)
                    def _():
                        stage_block(jn)
                    unpack(jn, b)
                    pltpu.async_copy(table.at[sv[b]], g[b], sg[b])
            return carry
        lax.fori_loop(0, NCH // 4, body, 0)
        pltpu.make_async_copy(table.at[sv[0]], g[0], sg[0]).wait()
        pltpu.async_copy(g[0], acc.at[dv[0]], ss[0], add=True)
        pltpu.make_async_copy(g[0], acc.at[dv[0]], ss[0]).wait()
        for b in range(1, 4):
            pltpu.make_async_copy(g[b], acc.at[dv[b]], ss[b]).wait()

    @pl.when(c == 0)
    def _():
        run(hL)

    @pl.when(c == 1)
    def _():
        run(hR)

    plsc.subcore_barrier()

    def copyout(out):
        def cchunk(kk, carry):
            t = s + kk * NS
            @pl.when(t < NZC)
            def _():
                pltpu.sync_copy(acc.at[pl.ds(t * ZK, ZK)],
                                out.at[pl.ds(t * ZK, ZK)])
            return carry
        lax.fori_loop(0, (NZC + NS - 1) // NS, cchunk, 0)

    @pl.when(c == 0)
    def _():
        copyout(outL)

    @pl.when(c == 1)
    def _():
        copyout(outR)


_sc_agg = functools.partial(
    pl.kernel,
    out_type=tuple(jax.ShapeDtypeStruct((N, HALF), jnp.float32)
                   for _ in range(2)),
    mesh=plsc.VectorSubcoreMesh(core_axis_name="c", subcore_axis_name="s"),
    scratch_types=[pltpu.VMEM((BLK, K), jnp.int32)]
    + [pltpu.VMEM((K,), jnp.int32) for _ in range(8)]
    + [pltpu.VMEM((K, HALF), jnp.float32) for _ in range(4)]
    + [pltpu.VMEM_SHARED((N, HALF), jnp.float32)]
    + [pltpu.SemaphoreType.DMA] * 8,
    compiler_params=pltpu.CompilerParams(use_tc_tiling_on_sc=False),
    name="sc_edge_segment_sum",
)(_sc_agg_body)


R = 2000


def _mlp_body(aL_ref, aR_ref, Wa_ref, ba_ref, g_ref, be_ref,
              Wb_ref, bb_ref, batch_ref, oL_ref, oR_ref, p_ref):
    z = jnp.concatenate([aL_ref[...], aR_ref[...]], axis=1)
    z = jnp.dot(z.astype(jnp.bfloat16), Wa_ref[...].astype(jnp.bfloat16),
                preferred_element_type=jnp.float32) + ba_ref[...]
    z = z * (g_ref[...] * INV_SQRT) + be_ref[...]
    z = jnp.maximum(z, 0.0)
    z = jnp.dot(z.astype(jnp.bfloat16), Wb_ref[...].astype(jnp.bfloat16),
                preferred_element_type=jnp.float32) + bb_ref[...]
    z = jnp.maximum(z, 0.0)
    oL_ref[...] = z[:, :HALF]
    oR_ref[...] = z[:, HALF:]
    onehot = (lax.broadcasted_iota(jnp.int32, (R, B), 1)
              == batch_ref[...]).astype(jnp.float32)
    pblk = lax.dot_general(onehot, z, (((0,), (0,)), ((), ())),
                           preferred_element_type=jnp.float32)
    @pl.when(pl.program_id(0) == 0)
    def _():
        p_ref[...] = jnp.zeros_like(p_ref)
    p_ref[...] += pblk


_mlp = pl.pallas_call(
    _mlp_body,
    grid=(N // R,),
    in_specs=(
        [pl.BlockSpec((R, HALF), lambda i: (i, 0)) for _ in range(2)]
        + [
            pl.BlockSpec((F, F), lambda i: (0, 0)),
            pl.BlockSpec((1, F), lambda i: (0, 0)),
            pl.BlockSpec((1, F), lambda i: (0, 0)),
            pl.BlockSpec((1, F), lambda i: (0, 0)),
            pl.BlockSpec((F, F), lambda i: (0, 0)),
            pl.BlockSpec((1, F), lambda i: (0, 0)),
            pl.BlockSpec((R, 1), lambda i: (i, 0)),
        ]
    ),
    out_specs=[pl.BlockSpec((R, HALF), lambda i: (i, 0)) for _ in range(2)]
    + [pl.BlockSpec((B, F), lambda i: (0, 0))],
    out_shape=[jax.ShapeDtypeStruct((N, HALF), jnp.float32) for _ in range(2)]
    + [jax.ShapeDtypeStruct((B, F), jnp.float32)],
)


def _final_body(p1_ref, p2_ref, p3_ref, lW1_ref, lb1_ref, lW2_ref, lb2_ref,
                out_ref):
    h = jnp.concatenate([p1_ref[...], p2_ref[...], p3_ref[...]], axis=1)
    h = jnp.dot(h, lW1_ref[...], preferred_element_type=jnp.float32) + lb1_ref[...]
    h = jnp.maximum(h, 0.0)
    out_ref[...] = (jnp.dot(h, lW2_ref[...], preferred_element_type=jnp.float32)
                    + lb2_ref[...])


_final = pl.pallas_call(
    _final_body,
    out_shape=jax.ShapeDtypeStruct((B, B), jnp.float32),
)


def kernel(x, edge_index, batch, W1a, b1a, g1, be1, W1b, b1b,
           W2a, b2a, g2, be2, W2b, b2b,
           W3a, b3a, g3, be3, W3b, b3b,
           lW1, lb1, lW2, lb2):
    pk3 = (edge_index[0] * 16384 + edge_index[1]).reshape(NS, NCH, K)
    batch2 = batch.reshape(N, 1)
    hq = (x[:, :HALF], x[:, HALF:])

    pools = []
    for (Wa, ba, g, be, Wb, bb) in ((W1a, b1a, g1, be1, W1b, b1b),
                                    (W2a, b2a, g2, be2, W2b, b2b),
                                    (W3a, b3a, g3, be3, W3b, b3b)):
        aq = _sc_agg(*hq, pk3)
        *hq, p = _mlp(*aq, Wa, ba.reshape(1, F), g.reshape(1, F),
                      be.reshape(1, F), Wb, bb.reshape(1, F), batch2)
        hq = tuple(hq)
        pools.append(p)

    return _final(pools[0], pools[1], pools[2], lW1, lb1.reshape(1, 3 * F),
                  lW2, lb2.reshape(1, B))


# speedup vs baseline: 1.0457x; 1.0457x over previous
import functools

import jax
import jax.numpy as jnp
import numpy as np
from jax import lax
from jax.experimental import pallas as pl
from jax.experimental.pallas import tpu as pltpu
from jax.experimental.pallas import tpu_sc as plsc

N = 10000
E = 160000
F = 256
HALF = 128
B = 64
NS = 16
EPT = E // NS
K = 80
NCH = EPT // K
ZK = 80
NZC = N // ZK
INV_SQRT = float(1.0 / np.sqrt(1.0 + 1e-5))


BLK = 25


def _sc_agg_body(hL, hR, pk3, outL, outR,
                 pkblk, s0, s1, s2, s3, d0, d1, d2, d3,
                 g0, g1, g2, g3, acc,
                 sg0, sg1, sg2, sg3, ss0, ss1, ss2, ss3, si):
    g = (g0, g1, g2, g3)
    sv = (s0, s1, s2, s3)
    dv = (d0, d1, d2, d3)
    sg = (sg0, sg1, sg2, sg3)
    ss = (ss0, ss1, ss2, ss3)
    c = lax.axis_index("c")
    s = lax.axis_index("s")

    def stage_block(jn):
        pltpu.sync_copy(pk3.at[s, pl.ds(jn, BLK)], pkblk)

    def unpack(jn, b):
        r = jn % BLK
        for t in range(K // 16):
            v = pkblk[r, pl.ds(t * 16, 16)]
            sv[b][pl.ds(t * 16, 16)] = lax.shift_right_logical(v, 14)
            dv[b][pl.ds(t * 16, 16)] = lax.bitwise_and(v, 16383)

    def acc_chunks(dma):
        def go(kk, carry):
            t = s + kk * NS
            @pl.when(t < NZC)
            def _():
                dma(pl.ds(t * ZK, ZK))
            return carry
        lax.fori_loop(0, (NZC + NS - 1) // NS, go, 0)

    def run(table):
        acc_chunks(lambda ds: pltpu.async_copy(table.at[ds], acc.at[ds], si))
        stage_block(0)
        for b in range(4):
            unpack(b, b)
            pltpu.async_copy(table.at[sv[b]], g[b], sg[b])
        acc_chunks(
            lambda ds: pltpu.make_async_copy(table.at[ds], acc.at[ds], si).wait())
        plsc.subcore_barrier()
        def body(jj, carry):
            j0 = 4 * jj
            for b in range(4):
                j = j0 + b
                pltpu.make_async_copy(table.at[sv[b]], g[b], sg[b]).wait()
                pltpu.async_copy(g[b], acc.at[dv[b]], ss[b], add=True)
                jn = j + 4
                @pl.when(jn < NCH)
                def _(b=b, jn=jn):
                    pltpu.make_async_copy(g[b], acc.at[dv[b]], ss[b]).wait()
                    @pl.when(jn % BLK == 0)
                    def _():
                        stage_block(jn)
                    unpack(jn, b)
                    pltpu.async_copy(table.at[sv[b]], g[b], sg[b])
            return carry
        lax.fori_loop(0, NCH // 4, body, 0)
        pltpu.make_async_copy(table.at[sv[0]], g[0], sg[0]).wait()
        pltpu.async_copy(g[0], acc.at[dv[0]], ss[0], add=True)
        pltpu.make_async_copy(g[0], acc.at[dv[0]], ss[0]).wait()
        for b in range(1, 4):
            pltpu.make_async_copy(g[b], acc.at[dv[b]], ss[b]).wait()
        plsc.subcore_barrier()

    def copyout(out):
        acc_chunks(lambda ds: pltpu.async_copy(acc.at[ds], out.at[ds], si))
        acc_chunks(
            lambda ds: pltpu.make_async_copy(acc.at[ds], out.at[ds], si).wait())

    @pl.when(c == 0)
    def _():
        run(hL)
        copyout(outL)

    @pl.when(c == 1)
    def _():
        run(hR)
        copyout(outR)


_sc_agg = functools.partial(
    pl.kernel,
    out_type=tuple(jax.ShapeDtypeStruct((N, HALF), jnp.float32)
                   for _ in range(2)),
    mesh=plsc.VectorSubcoreMesh(core_axis_name="c", subcore_axis_name="s"),
    scratch_types=[pltpu.VMEM((BLK, K), jnp.int32)]
    + [pltpu.VMEM((K,), jnp.int32) for _ in range(8)]
    + [pltpu.VMEM((K, HALF), jnp.float32) for _ in range(4)]
    + [pltpu.VMEM_SHARED((N, HALF), jnp.float32)]
    + [pltpu.SemaphoreType.DMA] * 9,
    compiler_params=pltpu.CompilerParams(use_tc_tiling_on_sc=False),
    name="sc_edge_segment_sum",
)(_sc_agg_body)


R = 2000


def _mlp_body(aL_ref, aR_ref, Wa_ref, ba_ref, g_ref, be_ref,
              Wb_ref, bb_ref, batch_ref, oL_ref, oR_ref, p_ref):
    z = jnp.concatenate([aL_ref[...], aR_ref[...]], axis=1)
    z = jnp.dot(z.astype(jnp.bfloat16), Wa_ref[...].astype(jnp.bfloat16),
                preferred_element_type=jnp.float32) + ba_ref[...]
    z = z * (g_ref[...] * INV_SQRT) + be_ref[...]
    z = jnp.maximum(z, 0.0)
    z = jnp.dot(z.astype(jnp.bfloat16), Wb_ref[...].astype(jnp.bfloat16),
                preferred_element_type=jnp.float32) + bb_ref[...]
    z = jnp.maximum(z, 0.0)
    oL_ref[...] = z[:, :HALF]
    oR_ref[...] = z[:, HALF:]
    onehot = (lax.broadcasted_iota(jnp.int32, (R, B), 1)
              == batch_ref[...]).astype(jnp.float32)
    pblk = lax.dot_general(onehot, z, (((0,), (0,)), ((), ())),
                           preferred_element_type=jnp.float32)
    @pl.when(pl.program_id(0) == 0)
    def _():
        p_ref[...] = jnp.zeros_like(p_ref)
    p_ref[...] += pblk


_mlp = pl.pallas_call(
    _mlp_body,
    grid=(N // R,),
    in_specs=(
        [pl.BlockSpec((R, HALF), lambda i: (i, 0)) for _ in range(2)]
        + [
            pl.BlockSpec((F, F), lambda i: (0, 0)),
            pl.BlockSpec((1, F), lambda i: (0, 0)),
            pl.BlockSpec((1, F), lambda i: (0, 0)),
            pl.BlockSpec((1, F), lambda i: (0, 0)),
            pl.BlockSpec((F, F), lambda i: (0, 0)),
            pl.BlockSpec((1, F), lambda i: (0, 0)),
            pl.BlockSpec((R, 1), lambda i: (i, 0)),
        ]
    ),
    out_specs=[pl.BlockSpec((R, HALF), lambda i: (i, 0)) for _ in range(2)]
    + [pl.BlockSpec((B, F), lambda i: (0, 0))],
    out_shape=[jax.ShapeDtypeStruct((N, HALF), jnp.float32) for _ in range(2)]
    + [jax.ShapeDtypeStruct((B, F), jnp.float32)],
)


def _final_body(p1_ref, p2_ref, p3_ref, lW1_ref, lb1_ref, lW2_ref, lb2_ref,
                out_ref):
    h = jnp.concatenate([p1_ref[...], p2_ref[...], p3_ref[...]], axis=1)
    h = jnp.dot(h, lW1_ref[...], preferred_element_type=jnp.float32) + lb1_ref[...]
    h = jnp.maximum(h, 0.0)
    out_ref[...] = (jnp.dot(h, lW2_ref[...], preferred_element_type=jnp.float32)
                    + lb2_ref[...])


_final = pl.pallas_call(
    _final_body,
    out_shape=jax.ShapeDtypeStruct((B, B), jnp.float32),
)


def kernel(x, edge_index, batch, W1a, b1a, g1, be1, W1b, b1b,
           W2a, b2a, g2, be2, W2b, b2b,
           W3a, b3a, g3, be3, W3b, b3b,
           lW1, lb1, lW2, lb2):
    pk3 = (edge_index[0] * 16384 + edge_index[1]).reshape(NS, NCH, K)
    batch2 = batch.reshape(N, 1)
    hq = (x[:, :HALF], x[:, HALF:])

    pools = []
    for (Wa, ba, g, be, Wb, bb) in ((W1a, b1a, g1, be1, W1b, b1b),
                                    (W2a, b2a, g2, be2, W2b, b2b),
                                    (W3a, b3a, g3, be3, W3b, b3b)):
        aq = _sc_agg(*hq, pk3)
        *hq, p = _mlp(*aq, Wa, ba.reshape(1, F), g.reshape(1, F),
                      be.reshape(1, F), Wb, bb.reshape(1, F), batch2)
        hq = tuple(hq)
        pools.append(p)

    return _final(pools[0], pools[1], pools[2], lW1, lb1.reshape(1, 3 * F),
                  lW2, lb2.reshape(1, B))


# speedup vs baseline: 1.0568x; 1.0106x over previous
import functools

import jax
import jax.numpy as jnp
import numpy as np
from jax import lax
from jax.experimental import pallas as pl
from jax.experimental.pallas import tpu as pltpu
from jax.experimental.pallas import tpu_sc as plsc

N = 10000
E = 160000
F = 256
HALF = 128
B = 64
NS = 16
EPT = E // NS
K = 80
NCH = EPT // K
ZK = 80
NZC = N // ZK
INV_SQRT = float(1.0 / np.sqrt(1.0 + 1e-5))


BLK = 25


def _sc_agg_body(hL, hR, pk3, outL, outR,
                 pkblk, s0, s1, s2, s3, d0, d1, d2, d3,
                 g0, g1, g2, g3, acc,
                 sg0, sg1, sg2, sg3, ss0, ss1, ss2, ss3, si, sk):
    g = (g0, g1, g2, g3)
    sv = (s0, s1, s2, s3)
    dv = (d0, d1, d2, d3)
    sg = (sg0, sg1, sg2, sg3)
    ss = (ss0, ss1, ss2, ss3)
    c = lax.axis_index("c")
    s = lax.axis_index("s")

    def stage_block(jn):
        h = ((jn // BLK) % 2) * BLK
        pltpu.async_copy(pk3.at[s, pl.ds(jn, BLK)], pkblk.at[pl.ds(h, BLK)], sk)

    def wait_block(jn):
        h = ((jn // BLK) % 2) * BLK
        pltpu.make_async_copy(pk3.at[s, pl.ds(jn, BLK)],
                              pkblk.at[pl.ds(h, BLK)], sk).wait()

    def unpack(jn, b):
        r = jn % (2 * BLK)
        for t in range(K // 16):
            v = pkblk[r, pl.ds(t * 16, 16)]
            sv[b][pl.ds(t * 16, 16)] = lax.shift_right_logical(v, 14)
            dv[b][pl.ds(t * 16, 16)] = lax.bitwise_and(v, 16383)

    def acc_chunks(dma):
        def go(kk, carry):
            t = s + kk * NS
            @pl.when(t < NZC)
            def _():
                dma(pl.ds(t * ZK, ZK))
            return carry
        lax.fori_loop(0, (NZC + NS - 1) // NS, go, 0)

    def run(table):
        acc_chunks(lambda ds: pltpu.async_copy(table.at[ds], acc.at[ds], si))
        stage_block(0)
        wait_block(0)
        stage_block(BLK)
        for b in range(4):
            unpack(b, b)
            pltpu.async_copy(table.at[sv[b]], g[b], sg[b])
        acc_chunks(
            lambda ds: pltpu.make_async_copy(table.at[ds], acc.at[ds], si).wait())
        plsc.subcore_barrier()
        def body(jj, carry):
            j0 = 4 * jj
            for b in range(4):
                j = j0 + b
                pltpu.make_async_copy(table.at[sv[b]], g[b], sg[b]).wait()
                pltpu.async_copy(g[b], acc.at[dv[b]], ss[b], add=True)
                jn = j + 4
                @pl.when(jn < NCH)
                def _(b=b, jn=jn):
                    pltpu.make_async_copy(g[b], acc.at[dv[b]], ss[b]).wait()
                    @pl.when(jn % BLK == 0)
                    def _():
                        wait_block(jn)
                        @pl.when(jn + BLK < NCH)
                        def _():
                            stage_block(jn + BLK)
                    unpack(jn, b)
                    pltpu.async_copy(table.at[sv[b]], g[b], sg[b])
            return carry
        lax.fori_loop(0, NCH // 4, body, 0)
        pltpu.make_async_copy(table.at[sv[0]], g[0], sg[0]).wait()
        pltpu.async_copy(g[0], acc.at[dv[0]], ss[0], add=True)
        pltpu.make_async_copy(g[0], acc.at[dv[0]], ss[0]).wait()
        for b in range(1, 4):
            pltpu.make_async_copy(g[b], acc.at[dv[b]], ss[b]).wait()
        plsc.subcore_barrier()

    def copyout(out):
        acc_chunks(lambda ds: pltpu.async_copy(acc.at[ds], out.at[ds], si))
        acc_chunks(
            lambda ds: pltpu.make_async_copy(acc.at[ds], out.at[ds], si).wait())

    @pl.when(c == 0)
    def _():
        run(hL)
        copyout(outL)

    @pl.when(c == 1)
    def _():
        run(hR)
        copyout(outR)


_sc_agg = functools.partial(
    pl.kernel,
    out_type=tuple(jax.ShapeDtypeStruct((N, HALF), jnp.float32)
                   for _ in range(2)),
    mesh=plsc.VectorSubcoreMesh(core_axis_name="c", subcore_axis_name="s"),
    scratch_types=[pltpu.VMEM((2 * BLK, K), jnp.int32)]
    + [pltpu.VMEM((K,), jnp.int32) for _ in range(8)]
    + [pltpu.VMEM((K, HALF), jnp.float32) for _ in range(4)]
    + [pltpu.VMEM_SHARED((N, HALF), jnp.float32)]
    + [pltpu.SemaphoreType.DMA] * 10,
    compiler_params=pltpu.CompilerParams(use_tc_tiling_on_sc=False),
    name="sc_edge_segment_sum",
)(_sc_agg_body)


R = 2000


def _mlp_body(aL_ref, aR_ref, WaL_ref, WaR_ref, ba_ref, g_ref, be_ref,
              Wb_ref, bb_ref, batch_ref, oL_ref, oR_ref, p_ref):
    z = (jnp.dot(aL_ref[...].astype(jnp.bfloat16),
                 WaL_ref[...].astype(jnp.bfloat16),
                 preferred_element_type=jnp.float32)
         + jnp.dot(aR_ref[...].astype(jnp.bfloat16),
                   WaR_ref[...].astype(jnp.bfloat16),
                   preferred_element_type=jnp.float32)
         + ba_ref[...])
    z = z * (g_ref[...] * INV_SQRT) + be_ref[...]
    z = jnp.maximum(z, 0.0)
    z = jnp.dot(z.astype(jnp.bfloat16), Wb_ref[...].astype(jnp.bfloat16),
                preferred_element_type=jnp.float32) + bb_ref[...]
    z = jnp.maximum(z, 0.0)
    oL_ref[...] = z[:, :HALF]
    oR_ref[...] = z[:, HALF:]
    onehot = (lax.broadcasted_iota(jnp.int32, (R, B), 1)
              == batch_ref[...]).astype(jnp.float32)
    pblk = lax.dot_general(onehot, z, (((0,), (0,)), ((), ())),
                           preferred_element_type=jnp.float32)
    @pl.when(pl.program_id(0) == 0)
    def _():
        p_ref[...] = jnp.zeros_like(p_ref)
    p_ref[...] += pblk


_mlp = pl.pallas_call(
    _mlp_body,
    grid=(N // R,),
    in_specs=(
        [pl.BlockSpec((R, HALF), lambda i: (i, 0)) for _ in range(2)]
        + [
            pl.BlockSpec((HALF, F), lambda i: (0, 0)),
            pl.BlockSpec((HALF, F), lambda i: (0, 0)),
            pl.BlockSpec((1, F), lambda i: (0, 0)),
            pl.BlockSpec((1, F), lambda i: (0, 0)),
            pl.BlockSpec((1, F), lambda i: (0, 0)),
            pl.BlockSpec((F, F), lambda i: (0, 0)),
            pl.BlockSpec((1, F), lambda i: (0, 0)),
            pl.BlockSpec((R, 1), lambda i: (i, 0)),
        ]
    ),
    out_specs=[pl.BlockSpec((R, HALF), lambda i: (i, 0)) for _ in range(2)]
    + [pl.BlockSpec((B, F), lambda i: (0, 0))],
    out_shape=[jax.ShapeDtypeStruct((N, HALF), jnp.float32) for _ in range(2)]
    + [jax.ShapeDtypeStruct((B, F), jnp.float32)],
)


def _final_body(p1_ref, p2_ref, p3_ref, lW1_ref, lb1_ref, lW2_ref, lb2_ref,
                out_ref):
    h = jnp.concatenate([p1_ref[...], p2_ref[...], p3_ref[...]], axis=1)
    h = jnp.dot(h, lW1_ref[...], preferred_element_type=jnp.float32) + lb1_ref[...]
    h = jnp.maximum(h, 0.0)
    out_ref[...] = (jnp.dot(h, lW2_ref[...], preferred_element_type=jnp.float32)
                    + lb2_ref[...])


_final = pl.pallas_call(
    _final_body,
    out_shape=jax.ShapeDtypeStruct((B, B), jnp.float32),
)


def kernel(x, edge_index, batch, W1a, b1a, g1, be1, W1b, b1b,
           W2a, b2a, g2, be2, W2b, b2b,
           W3a, b3a, g3, be3, W3b, b3b,
           lW1, lb1, lW2, lb2):
    pk3 = (edge_index[0] * 16384 + edge_index[1]).reshape(NS, NCH, K)
    batch2 = batch.reshape(N, 1)
    hq = (x[:, :HALF], x[:, HALF:])

    pools = []
    for (Wa, ba, g, be, Wb, bb) in ((W1a, b1a, g1, be1, W1b, b1b),
                                    (W2a, b2a, g2, be2, W2b, b2b),
                                    (W3a, b3a, g3, be3, W3b, b3b)):
        aq = _sc_agg(*hq, pk3)
        *hq, p = _mlp(*aq, Wa[:HALF], Wa[HALF:], ba.reshape(1, F),
                      g.reshape(1, F), be.reshape(1, F), Wb,
                      bb.reshape(1, F), batch2)
        hq = tuple(hq)
        pools.append(p)

    return _final(pools[0], pools[1], pools[2], lW1, lb1.reshape(1, 3 * F),
                  lW2, lb2.reshape(1, B))


# speedup vs baseline: 1.0571x; 1.0002x over previous
import functools

import jax
import jax.numpy as jnp
import numpy as np
from jax import lax
from jax.experimental import pallas as pl
from jax.experimental.pallas import tpu as pltpu
from jax.experimental.pallas import tpu_sc as plsc

N = 10000
E = 160000
F = 256
HALF = 128
B = 64
NS = 16
EPT = E // NS
K = 80
NCH = EPT // K
ZK = 80
NZC = N // ZK
INV_SQRT = float(1.0 / np.sqrt(1.0 + 1e-5))


BLK = 25


def _sc_agg_body(hL, hR, pk3, outL, outR,
                 pkblk, s0, s1, s2, s3, d0, d1, d2, d3,
                 g0, g1, g2, g3, acc,
                 sg0, sg1, sg2, sg3, ss0, ss1, ss2, ss3, si, sk):
    g = (g0, g1, g2, g3)
    sv = (s0, s1, s2, s3)
    dv = (d0, d1, d2, d3)
    sg = (sg0, sg1, sg2, sg3)
    ss = (ss0, ss1, ss2, ss3)
    c = lax.axis_index("c")
    s = lax.axis_index("s")

    def stage_block(jn):
        h = ((jn // BLK) % 2) * BLK
        pltpu.async_copy(pk3.at[s, pl.ds(jn, BLK)], pkblk.at[pl.ds(h, BLK)], sk)

    def wait_block(jn):
        h = ((jn // BLK) % 2) * BLK
        pltpu.make_async_copy(pk3.at[s, pl.ds(jn, BLK)],
                              pkblk.at[pl.ds(h, BLK)], sk).wait()

    def unpack(jn, b):
        r = jn % (2 * BLK)
        for t in range(K // 16):
            v = pkblk[r, pl.ds(t * 16, 16)]
            sv[b][pl.ds(t * 16, 16)] = lax.shift_right_logical(v, 14)
            dv[b][pl.ds(t * 16, 16)] = lax.bitwise_and(v, 16383)

    def acc_chunks(dma):
        def go(kk, carry):
            t = s + kk * NS
            @pl.when(t < NZC)
            def _():
                dma(pl.ds(t * ZK, ZK))
            return carry
        lax.fori_loop(0, (NZC + NS - 1) // NS, go, 0)

    def run(table):
        acc_chunks(lambda ds: pltpu.async_copy(table.at[ds], acc.at[ds], si))
        stage_block(0)
        wait_block(0)
        stage_block(BLK)
        for b in range(4):
            unpack(b, b)
            pltpu.async_copy(table.at[sv[b]], g[b], sg[b])
        acc_chunks(
            lambda ds: pltpu.make_async_copy(table.at[ds], acc.at[ds], si).wait())
        plsc.subcore_barrier()
        def body(jj, carry):
            j0 = 4 * jj
            for b in range(4):
                j = j0 + b
                pltpu.make_async_copy(table.at[sv[b]], g[b], sg[b]).wait()
                pltpu.async_copy(g[b], acc.at[dv[b]], ss[b], add=True)
                jn = j + 4
                @pl.when(jn < NCH)
                def _(b=b, jn=jn):
                    pltpu.make_async_copy(g[b], acc.at[dv[b]], ss[b]).wait()
                    @pl.when(jn % BLK == 0)
                    def _():
                        wait_block(jn)
                        @pl.when(jn + BLK < NCH)
                        def _():
                            stage_block(jn + BLK)
                    unpack(jn, b)
                    pltpu.async_copy(table.at[sv[b]], g[b], sg[b])
            return carry
        lax.fori_loop(0, NCH // 4, body, 0)
        pltpu.make_async_copy(table.at[sv[0]], g[0], sg[0]).wait()
        pltpu.async_copy(g[0], acc.at[dv[0]], ss[0], add=True)
        pltpu.make_async_copy(g[0], acc.at[dv[0]], ss[0]).wait()
        for b in range(1, 4):
            pltpu.make_async_copy(g[b], acc.at[dv[b]], ss[b]).wait()
        plsc.subcore_barrier()

    def copyout(out):
        acc_chunks(lambda ds: pltpu.async_copy(acc.at[ds], out.at[ds], si))
        acc_chunks(
            lambda ds: pltpu.make_async_copy(acc.at[ds], out.at[ds], si).wait())

    @pl.when(c == 0)
    def _():
        run(hL)
        copyout(outL)

    @pl.when(c == 1)
    def _():
        run(hR)
        copyout(outR)


_sc_agg = functools.partial(
    pl.kernel,
    out_type=tuple(jax.ShapeDtypeStruct((N, HALF), jnp.float32)
                   for _ in range(2)),
    mesh=plsc.VectorSubcoreMesh(core_axis_name="c", subcore_axis_name="s"),
    scratch_types=[pltpu.VMEM((2 * BLK, K), jnp.int32)]
    + [pltpu.VMEM((K,), jnp.int32) for _ in range(8)]
    + [pltpu.VMEM((K, HALF), jnp.float32) for _ in range(4)]
    + [pltpu.VMEM_SHARED((N, HALF), jnp.float32)]
    + [pltpu.SemaphoreType.DMA] * 10,
    compiler_params=pltpu.CompilerParams(use_tc_tiling_on_sc=False),
    name="sc_edge_segment_sum",
)(_sc_agg_body)


R = 2000


def _mlp_body(aL_ref, aR_ref, WaL_ref, WaR_ref, ba_ref, g_ref, be_ref,
              Wb_ref, bb_ref, batch_ref, oL_ref, oR_ref, p_ref):
    z = (jnp.dot(aL_ref[...].astype(jnp.bfloat16),
                 WaL_ref[...].astype(jnp.bfloat16),
                 preferred_element_type=jnp.float32)
         + jnp.dot(aR_ref[...].astype(jnp.bfloat16),
                   WaR_ref[...].astype(jnp.bfloat16),
                   preferred_element_type=jnp.float32)
         + ba_ref[...])
    z = z * (g_ref[...] * INV_SQRT) + be_ref[...]
    z = jnp.maximum(z, 0.0)
    z = jnp.dot(z.astype(jnp.bfloat16), Wb_ref[...].astype(jnp.bfloat16),
                preferred_element_type=jnp.float32) + bb_ref[...]
    z = jnp.maximum(z, 0.0)
    oL_ref[...] = z[:, :HALF]
    oR_ref[...] = z[:, HALF:]
    onehot = (lax.broadcasted_iota(jnp.int32, (R, B), 1)
              == batch_ref[...]).astype(jnp.float32)
    pblk = lax.dot_general(onehot, z, (((0,), (0,)), ((), ())),
                           preferred_element_type=jnp.float32)
    @pl.when(pl.program_id(0) == 0)
    def _():
        p_ref[...] = jnp.zeros_like(p_ref)
    p_ref[...] += pblk


_mlp = pl.pallas_call(
    _mlp_body,
    grid=(N // R,),
    in_specs=(
        [pl.BlockSpec((R, HALF), lambda i: (i, 0)) for _ in range(2)]
        + [
            pl.BlockSpec((HALF, F), lambda i: (0, 0)),
            pl.BlockSpec((HALF, F), lambda i: (0, 0)),
            pl.BlockSpec((1, F), lambda i: (0, 0)),
            pl.BlockSpec((1, F), lambda i: (0, 0)),
            pl.BlockSpec((1, F), lambda i: (0, 0)),
            pl.BlockSpec((F, F), lambda i: (0, 0)),
            pl.BlockSpec((1, F), lambda i: (0, 0)),
            pl.BlockSpec((R, 1), lambda i: (i, 0)),
        ]
    ),
    out_specs=[pl.BlockSpec((R, HALF), lambda i: (i, 0)) for _ in range(2)]
    + [pl.BlockSpec((B, F), lambda i: (0, 0))],
    out_shape=[jax.ShapeDtypeStruct((N, HALF), jnp.float32) for _ in range(2)]
    + [jax.ShapeDtypeStruct((B, F), jnp.float32)],
)


def _mlp_last_body(aL_ref, aR_ref, WaL_ref, WaR_ref, ba_ref, g_ref, be_ref,
                   Wb_ref, bb_ref, batch_ref, p1_ref, p2_ref,
                   lW1_ref, lb1_ref, lW2_ref, lb2_ref,
                   oL_ref, oR_ref, p_ref, out_ref):
    _mlp_body(aL_ref, aR_ref, WaL_ref, WaR_ref, ba_ref, g_ref, be_ref,
              Wb_ref, bb_ref, batch_ref, oL_ref, oR_ref, p_ref)
    @pl.when(pl.program_id(0) == N // R - 1)
    def _():
        h = jnp.concatenate([p1_ref[...], p2_ref[...], p_ref[...]], axis=1)
        h = (jnp.dot(h, lW1_ref[...], preferred_element_type=jnp.float32)
             + lb1_ref[...])
        h = jnp.maximum(h, 0.0)
        out_ref[...] = (jnp.dot(h, lW2_ref[...],
                                preferred_element_type=jnp.float32)
                        + lb2_ref[...])


_mlp_last = pl.pallas_call(
    _mlp_last_body,
    grid=(N // R,),
    in_specs=(
        [pl.BlockSpec((R, HALF), lambda i: (i, 0)) for _ in range(2)]
        + [
            pl.BlockSpec((HALF, F), lambda i: (0, 0)),
            pl.BlockSpec((HALF, F), lambda i: (0, 0)),
            pl.BlockSpec((1, F), lambda i: (0, 0)),
            pl.BlockSpec((1, F), lambda i: (0, 0)),
            pl.BlockSpec((1, F), lambda i: (0, 0)),
            pl.BlockSpec((F, F), lambda i: (0, 0)),
            pl.BlockSpec((1, F), lambda i: (0, 0)),
            pl.BlockSpec((R, 1), lambda i: (i, 0)),
            pl.BlockSpec((B, F), lambda i: (0, 0)),
            pl.BlockSpec((B, F), lambda i: (0, 0)),
            pl.BlockSpec((3 * F, 3 * F), lambda i: (0, 0)),
            pl.BlockSpec((1, 3 * F), lambda i: (0, 0)),
            pl.BlockSpec((3 * F, B), lambda i: (0, 0)),
            pl.BlockSpec((1, B), lambda i: (0, 0)),
        ]
    ),
    out_specs=[pl.BlockSpec((R, HALF), lambda i: (i, 0)) for _ in range(2)]
    + [pl.BlockSpec((B, F), lambda i: (0, 0)),
       pl.BlockSpec((B, B), lambda i: (0, 0))],
    out_shape=[jax.ShapeDtypeStruct((N, HALF), jnp.float32) for _ in range(2)]
    + [jax.ShapeDtypeStruct((B, F), jnp.float32),
       jax.ShapeDtypeStruct((B, B), jnp.float32)],
)


def kernel(x, edge_index, batch, W1a, b1a, g1, be1, W1b, b1b,
           W2a, b2a, g2, be2, W2b, b2b,
           W3a, b3a, g3, be3, W3b, b3b,
           lW1, lb1, lW2, lb2):
    pk3 = (edge_index[0] * 16384 + edge_index[1]).reshape(NS, NCH, K)
    batch2 = batch.reshape(N, 1)
    hq = (x[:, :HALF], x[:, HALF:])

    pools = []
    for (Wa, ba, g, be, Wb, bb) in ((W1a, b1a, g1, be1, W1b, b1b),
                                    (W2a, b2a, g2, be2, W2b, b2b)):
        aq = _sc_agg(*hq, pk3)
        *hq, p = _mlp(*aq, Wa[:HALF], Wa[HALF:], ba.reshape(1, F),
                      g.reshape(1, F), be.reshape(1, F), Wb,
                      bb.reshape(1, F), batch2)
        hq = tuple(hq)
        pools.append(p)

    aq = _sc_agg(*hq, pk3)
    _, _, _, out = _mlp_last(
        *aq, W3a[:HALF], W3a[HALF:], b3a.reshape(1, F), g3.reshape(1, F),
        be3.reshape(1, F), W3b, b3b.reshape(1, F), batch2,
        pools[0], pools[1], lW1, lb1.reshape(1, 3 * F), lW2,
        lb2.reshape(1, B))
    return out


# speedup vs baseline: 1.0632x; 1.0058x over previous
import functools

import jax
import jax.numpy as jnp
import numpy as np
from jax import lax
from jax.experimental import pallas as pl
from jax.experimental.pallas import tpu as pltpu
from jax.experimental.pallas import tpu_sc as plsc

N = 10000
E = 160000
F = 256
HALF = 128
B = 64
NS = 16
EPT = E // NS
K = 80
NCH = EPT // K
ZK = 80
NZC = N // ZK
INV_SQRT = float(1.0 / np.sqrt(1.0 + 1e-5))


BLK = 25


def _sc_agg_body(hL, hR, pk3, outL, outR,
                 pkblk, s0, s1, s2, s3, d0, d1, d2, d3,
                 g0, g1, g2, g3, acc,
                 sg0, sg1, sg2, sg3, ss0, ss1, ss2, ss3, si, sk):
    g = (g0, g1, g2, g3)
    sv = (s0, s1, s2, s3)
    dv = (d0, d1, d2, d3)
    sg = (sg0, sg1, sg2, sg3)
    ss = (ss0, ss1, ss2, ss3)
    c = lax.axis_index("c")
    s = lax.axis_index("s")

    def stage_block(jn):
        h = ((jn // BLK) % 2) * BLK
        pltpu.async_copy(pk3.at[s, pl.ds(jn, BLK)], pkblk.at[pl.ds(h, BLK)], sk)

    def wait_block(jn):
        h = ((jn // BLK) % 2) * BLK
        pltpu.make_async_copy(pk3.at[s, pl.ds(jn, BLK)],
                              pkblk.at[pl.ds(h, BLK)], sk).wait()

    def unpack(jn, b):
        r = jn % (2 * BLK)
        for t in range(K // 16):
            v = pkblk[r, pl.ds(t * 16, 16)]
            sv[b][pl.ds(t * 16, 16)] = lax.shift_right_logical(v, 14)
            dv[b][pl.ds(t * 16, 16)] = lax.bitwise_and(v, 16383)

    def acc_chunks(dma):
        def go(kk, carry):
            t = s + kk * NS
            @pl.when(t < NZC)
            def _():
                dma(pl.ds(t * ZK, ZK))
            return carry
        lax.fori_loop(0, (NZC + NS - 1) // NS, go, 0)

    def run(table):
        acc_chunks(lambda ds: pltpu.async_copy(table.at[ds], acc.at[ds], si))
        stage_block(0)
        wait_block(0)
        stage_block(BLK)
        for b in range(4):
            unpack(b, b)
            pltpu.async_copy(table.at[sv[b]], g[b], sg[b])
        acc_chunks(
            lambda ds: pltpu.make_async_copy(table.at[ds], acc.at[ds], si).wait())
        plsc.subcore_barrier()
        def body(jj, carry):
            j0 = 4 * jj
            for b in range(4):
                j = j0 + b
                pltpu.make_async_copy(table.at[sv[b]], g[b], sg[b]).wait()
                pltpu.async_copy(g[b], acc.at[dv[b]], ss[b], add=True)
                jn = j + 4
                @pl.when(jn < NCH)
                def _(b=b, jn=jn):
                    pltpu.make_async_copy(g[b], acc.at[dv[b]], ss[b]).wait()
                    @pl.when(jn % BLK == 0)
                    def _():
                        wait_block(jn)
                        @pl.when(jn + BLK < NCH)
                        def _():
                            stage_block(jn + BLK)
                    unpack(jn, b)
                    pltpu.async_copy(table.at[sv[b]], g[b], sg[b])
            return carry
        lax.fori_loop(0, NCH // 4, body, 0)
        pltpu.make_async_copy(table.at[sv[0]], g[0], sg[0]).wait()
        pltpu.async_copy(g[0], acc.at[dv[0]], ss[0], add=True)
        pltpu.make_async_copy(g[0], acc.at[dv[0]], ss[0]).wait()
        for b in range(1, 4):
            pltpu.make_async_copy(g[b], acc.at[dv[b]], ss[b]).wait()
        plsc.subcore_barrier()

    def copyout(out):
        acc_chunks(lambda ds: pltpu.async_copy(acc.at[ds], out.at[ds], si))
        acc_chunks(
            lambda ds: pltpu.make_async_copy(acc.at[ds], out.at[ds], si).wait())

    @pl.when(c == 0)
    def _():
        run(hL)
        copyout(outL)

    @pl.when(c == 1)
    def _():
        run(hR)
        copyout(outR)


_sc_agg = functools.partial(
    pl.kernel,
    out_type=tuple(jax.ShapeDtypeStruct((N, HALF), jnp.float32)
                   for _ in range(2)),
    mesh=plsc.VectorSubcoreMesh(core_axis_name="c", subcore_axis_name="s"),
    scratch_types=[pltpu.VMEM((2 * BLK, K), jnp.int32)]
    + [pltpu.VMEM((K,), jnp.int32) for _ in range(8)]
    + [pltpu.VMEM((K, HALF), jnp.float32) for _ in range(4)]
    + [pltpu.VMEM_SHARED((N, HALF), jnp.float32)]
    + [pltpu.SemaphoreType.DMA] * 10,
    compiler_params=pltpu.CompilerParams(use_tc_tiling_on_sc=False),
    name="sc_edge_segment_sum",
)(_sc_agg_body)


R = 5000


def _mlp_body(aL_ref, aR_ref, WaL_ref, WaR_ref, ba_ref, g_ref, be_ref,
              Wb_ref, bb_ref, batch_ref, oL_ref, oR_ref, p_ref):
    z = (jnp.dot(aL_ref[...].astype(jnp.bfloat16),
                 WaL_ref[...].astype(jnp.bfloat16),
                 preferred_element_type=jnp.float32)
         + jnp.dot(aR_ref[...].astype(jnp.bfloat16),
                   WaR_ref[...].astype(jnp.bfloat16),
                   preferred_element_type=jnp.float32)
         + ba_ref[...])
    z = z * (g_ref[...] * INV_SQRT) + be_ref[...]
    z = jnp.maximum(z, 0.0)
    z = jnp.dot(z.astype(jnp.bfloat16), Wb_ref[...].astype(jnp.bfloat16),
                preferred_element_type=jnp.float32) + bb_ref[...]
    z = jnp.maximum(z, 0.0)
    oL_ref[...] = z[:, :HALF]
    oR_ref[...] = z[:, HALF:]
    onehot = (lax.broadcasted_iota(jnp.int32, (R, B), 1)
              == batch_ref[...]).astype(jnp.float32)
    pblk = lax.dot_general(onehot, z, (((0,), (0,)), ((), ())),
                           preferred_element_type=jnp.float32)
    @pl.when(pl.program_id(0) == 0)
    def _():
        p_ref[...] = jnp.zeros_like(p_ref)
    p_ref[...] += pblk


_mlp = pl.pallas_call(
    _mlp_body,
    grid=(N // R,),
    in_specs=(
        [pl.BlockSpec((R, HALF), lambda i: (i, 0)) for _ in range(2)]
        + [
            pl.BlockSpec((HALF, F), lambda i: (0, 0)),
            pl.BlockSpec((HALF, F), lambda i: (0, 0)),
            pl.BlockSpec((1, F), lambda i: (0, 0)),
            pl.BlockSpec((1, F), lambda i: (0, 0)),
            pl.BlockSpec((1, F), lambda i: (0, 0)),
            pl.BlockSpec((F, F), lambda i: (0, 0)),
            pl.BlockSpec((1, F), lambda i: (0, 0)),
            pl.BlockSpec((R, 1), lambda i: (i, 0)),
        ]
    ),
    out_specs=[pl.BlockSpec((R, HALF), lambda i: (i, 0)) for _ in range(2)]
    + [pl.BlockSpec((B, F), lambda i: (0, 0))],
    out_shape=[jax.ShapeDtypeStruct((N, HALF), jnp.float32) for _ in range(2)]
    + [jax.ShapeDtypeStruct((B, F), jnp.float32)],
)


def _mlp_last_body(aL_ref, aR_ref, WaL_ref, WaR_ref, ba_ref, g_ref, be_ref,
                   Wb_ref, bb_ref, batch_ref, p1_ref, p2_ref,
                   lW1_ref, lb1_ref, lW2_ref, lb2_ref,
                   oL_ref, oR_ref, p_ref, out_ref):
    _mlp_body(aL_ref, aR_ref, WaL_ref, WaR_ref, ba_ref, g_ref, be_ref,
              Wb_ref, bb_ref, batch_ref, oL_ref, oR_ref, p_ref)
    @pl.when(pl.program_id(0) == N // R - 1)
    def _():
        h = jnp.concatenate([p1_ref[...], p2_ref[...], p_ref[...]], axis=1)
        h = (jnp.dot(h, lW1_ref[...], preferred_element_type=jnp.float32)
             + lb1_ref[...])
        h = jnp.maximum(h, 0.0)
        out_ref[...] = (jnp.dot(h, lW2_ref[...],
                                preferred_element_type=jnp.float32)
                        + lb2_ref[...])


_mlp_last = pl.pallas_call(
    _mlp_last_body,
    grid=(N // R,),
    in_specs=(
        [pl.BlockSpec((R, HALF), lambda i: (i, 0)) for _ in range(2)]
        + [
            pl.BlockSpec((HALF, F), lambda i: (0, 0)),
            pl.BlockSpec((HALF, F), lambda i: (0, 0)),
            pl.BlockSpec((1, F), lambda i: (0, 0)),
            pl.BlockSpec((1, F), lambda i: (0, 0)),
            pl.BlockSpec((1, F), lambda i: (0, 0)),
            pl.BlockSpec((F, F), lambda i: (0, 0)),
            pl.BlockSpec((1, F), lambda i: (0, 0)),
            pl.BlockSpec((R, 1), lambda i: (i, 0)),
            pl.BlockSpec((B, F), lambda i: (0, 0)),
            pl.BlockSpec((B, F), lambda i: (0, 0)),
            pl.BlockSpec((3 * F, 3 * F), lambda i: (0, 0)),
            pl.BlockSpec((1, 3 * F), lambda i: (0, 0)),
            pl.BlockSpec((3 * F, B), lambda i: (0, 0)),
            pl.BlockSpec((1, B), lambda i: (0, 0)),
        ]
    ),
    out_specs=[pl.BlockSpec((R, HALF), lambda i: (i, 0)) for _ in range(2)]
    + [pl.BlockSpec((B, F), lambda i: (0, 0)),
       pl.BlockSpec((B, B), lambda i: (0, 0))],
    out_shape=[jax.ShapeDtypeStruct((N, HALF), jnp.float32) for _ in range(2)]
    + [jax.ShapeDtypeStruct((B, F), jnp.float32),
       jax.ShapeDtypeStruct((B, B), jnp.float32)],
)


def kernel(x, edge_index, batch, W1a, b1a, g1, be1, W1b, b1b,
           W2a, b2a, g2, be2, W2b, b2b,
           W3a, b3a, g3, be3, W3b, b3b,
           lW1, lb1, lW2, lb2):
    pk3 = (edge_index[0] * 16384 + edge_index[1]).reshape(NS, NCH, K)
    batch2 = batch.reshape(N, 1)
    hq = (x[:, :HALF], x[:, HALF:])

    pools = []
    for (Wa, ba, g, be, Wb, bb) in ((W1a, b1a, g1, be1, W1b, b1b),
                                    (W2a, b2a, g2, be2, W2b, b2b)):
        aq = _sc_agg(*hq, pk3)
        *hq, p = _mlp(*aq, Wa[:HALF], Wa[HALF:], ba.reshape(1, F),
                      g.reshape(1, F), be.reshape(1, F), Wb,
                      bb.reshape(1, F), batch2)
        hq = tuple(hq)
        pools.append(p)

    aq = _sc_agg(*hq, pk3)
    _, _, _, out = _mlp_last(
        *aq, W3a[:HALF], W3a[HALF:], b3a.reshape(1, F), g3.reshape(1, F),
        be3.reshape(1, F), W3b, b3b.reshape(1, F), batch2,
        pools[0], pools[1], lW1, lb1.reshape(1, 3 * F), lW2,
        lb2.reshape(1, B))
    return out
